# baseline (speedup 1.0000x reference)
import functools

import jax
import jax.numpy as jnp
import numpy as np
from jax import lax
from jax.experimental import pallas as pl
from jax.experimental.pallas import tpu as pltpu
from jax.experimental.pallas import tpu_sc as plsc

N = 10000
NPAD = 10240
NW = 32
CK = 128
H = 64
RPT_DEG = NPAD // 16
RPT_AGG = (NPAD // 2) // 16


def _mesh():
    return plsc.VectorSubcoreMesh(core_axis_name="c", subcore_axis_name="s")


def _deg_body(dsts_hbm, const_hbm, out_hbm, idx_dst, ones_v, acc, sem):
    c = lax.axis_index("c")
    s = lax.axis_index("s")
    wid = s * 2 + c
    nchunks = dsts_hbm.shape[1]
    pltpu.sync_copy(const_hbm.at[pl.ds(0, CK)], ones_v)
    pltpu.sync_copy(dsts_hbm.at[wid], idx_dst)
    pltpu.sync_copy(
        const_hbm.at[pl.ds(CK, RPT_DEG)],
        acc.at[pl.ds(s * RPT_DEG, RPT_DEG)],
    )
    plsc.subcore_barrier()

    def step(j, _):
        pltpu.async_copy(ones_v, acc.at[idx_dst.at[j]], sem, add=True)
        return 0

    lax.fori_loop(0, nchunks, step, 0)

    def drain(j, _):
        pltpu.make_async_copy(ones_v, acc.at[idx_dst.at[j]], sem).wait()
        return 0

    lax.fori_loop(0, nchunks, drain, 0)
    plsc.subcore_barrier()
    pltpu.sync_copy(
        acc.at[pl.ds(s * RPT_DEG, RPT_DEG)],
        out_hbm.at[c].at[pl.ds(s * RPT_DEG, RPT_DEG)],
    )


def _sc_deg(dsts, const16):
    kern = pl.kernel(
        _deg_body,
        out_type=jax.ShapeDtypeStruct((2, NPAD, 16), jnp.float32),
        mesh=_mesh(),
        scratch_types=[
            pltpu.VMEM(dsts.shape[1:], jnp.int32),
            pltpu.VMEM((CK, 16), jnp.float32),
            pltpu.VMEM_SHARED((NPAD, 16), jnp.float32),
            pltpu.SemaphoreType.DMA,
        ],
    )
    return kern(dsts, const16)


NBUF = 8
LOOK = 4


def _agg_body(tbl_hbm, gsrcs_hbm, sdsts_hbm, zeros_hbm, out_hbm,
              idx_src, idx_dst, rows, acc, *sems):
    c = lax.axis_index("c")
    s = lax.axis_index("s")
    wid = s * 2 + c
    nchunks = gsrcs_hbm.shape[1]
    gsems, ssems = sems[:NBUF], sems[NBUF:]
    pltpu.sync_copy(gsrcs_hbm.at[wid], idx_src)
    pltpu.sync_copy(sdsts_hbm.at[wid], idx_dst)
    pltpu.sync_copy(zeros_hbm, acc.at[pl.ds(s * RPT_DEG, RPT_DEG)])
    plsc.subcore_barrier()

    def gather(j, b):
        pltpu.async_copy(tbl_hbm.at[idx_src.at[j]], rows.at[b], gsems[b])

    def wait_gather(j, b):
        pltpu.make_async_copy(tbl_hbm.at[idx_src.at[j]], rows.at[b],
                              gsems[b]).wait()

    def scatter(j, b):
        pltpu.async_copy(rows.at[b], acc.at[idx_dst.at[j]], ssems[b],
                         add=True)

    def wait_scatter(j, b):
        pltpu.make_async_copy(rows.at[b], acc.at[idx_dst.at[j]],
                              ssems[b]).wait()

    for b in range(LOOK):
        gather(b, b)

    def outer(i, _):
        j0 = NBUF * i
        for b in range(NBUF):
            j = j0 + b
            wait_gather(j, b)
            scatter(j, b)
            bn = (b + LOOK) % NBUF
            @pl.when(j + LOOK < nchunks)
            def _(j=j, bn=bn):
                @pl.when(j - LOOK >= 0)
                def _():
                    wait_scatter(j - LOOK, bn)
                gather(j + LOOK, bn)
        return 0

    lax.fori_loop(0, nchunks // NBUF, outer, 0)
    for k in range(NBUF):
        j = nchunks - NBUF + k
        wait_scatter(j, j % NBUF)
    plsc.subcore_barrier()
    pltpu.sync_copy(
        acc.at[pl.ds(s * RPT_DEG, RPT_DEG)],
        out_hbm.at[c].at[pl.ds(s * RPT_DEG, RPT_DEG)],
    )


def _sc_agg(tbl, gsrcs, sdsts, zeros):
    kern = pl.kernel(
        _agg_body,
        out_type=jax.ShapeDtypeStruct((2, NPAD, H), jnp.float32),
        mesh=_mesh(),
        scratch_types=[
            pltpu.VMEM(gsrcs.shape[1:], jnp.int32),
            pltpu.VMEM(sdsts.shape[1:], jnp.int32),
            pltpu.VMEM((NBUF, CK, H), jnp.float32),
            pltpu.VMEM_SHARED((NPAD, H), jnp.float32),
        ] + [pltpu.SemaphoreType.DMA] * (2 * NBUF),
        compiler_params=pltpu.CompilerParams(use_tc_tiling_on_sc=False),
    )
    return kern(tbl, gsrcs, sdsts, zeros)


def _decode_body(zi_ref, zj_ref, out_ref):
    logits = jax.lax.dot_general(
        zi_ref[...], zj_ref[...],
        (((1,), (1,)), ((), ())),
        preferred_element_type=jnp.float32,
    )
    out_ref[...] = 0.5 * (1.0 + jnp.tanh(logits * 0.5))


@functools.partial(jax.jit, static_argnames=("bm",))
def _decode(z, bm=128):
    n, cdim = z.shape
    grid = (pl.cdiv(n, bm),)
    return pl.pallas_call(
        _decode_body,
        grid=grid,
        in_specs=[
            pl.BlockSpec((bm, cdim), lambda i: (i, 0)),
            pl.BlockSpec((n, cdim), lambda i: (0, 0)),
        ],
        out_specs=pl.BlockSpec((bm, n), lambda i: (i, 0)),
        out_shape=jax.ShapeDtypeStruct((n, n), jnp.float32),
    )(z, z)


def _prep_idx(idx, pad_base, n_pad_zone_rows=32):
    e = idx.shape[0]
    epw = -(-e // NW)
    nchunks = -(-epw // CK)
    nchunks = -(-nchunks // NBUF) * NBUF
    total = NW * nchunks * CK
    pad = jnp.asarray(
        pad_base + (np.arange(total - e) % n_pad_zone_rows), jnp.int32)
    return jnp.concatenate([idx.astype(jnp.int32), pad]).reshape(
        NW, nchunks, CK)


def kernel(x, edge_index, eps, W1, b1, Wmu, bmu, Wls, bls):
    src, dst = edge_index[0], edge_index[1]
    src = src.astype(jnp.int32)
    dst = dst.astype(jnp.int32)
    dsts = _prep_idx(dst, N)
    gsrcs = _prep_idx(src, 0)
    sdsts = dsts
    zeros = jnp.zeros((RPT_DEG, H), jnp.float32)
    const16 = jnp.concatenate(
        [jnp.ones((CK, 16), jnp.float32),
         jnp.zeros((RPT_DEG, 16), jnp.float32)])

    degp = _sc_deg(dsts, const16)
    deg = degp[0, :N, 0] + degp[1, :N, 0] + 1.0
    dinv = jax.lax.rsqrt(deg)

    def prop(g):
        parts = _sc_agg(g, gsrcs, sdsts, zeros)
        return parts[0, :N] + parts[1, :N] + g

    g1 = (x @ W1) * dinv[:, None]
    h = jax.nn.relu(prop(g1) * dinv[:, None] + b1)

    g2 = h * dinv[:, None]
    q = prop(g2) * dinv[:, None]

    mu = q @ Wmu + bmu
    logstd = q @ Wls + bls
    z = mu + eps * jnp.exp(logstd)
    adj = _decode(z)
    return adj, mu, logstd


# speedup vs baseline: 1.0130x; 1.0130x over previous
import functools

import jax
import jax.numpy as jnp
import numpy as np
from jax import lax
from jax.experimental import pallas as pl
from jax.experimental.pallas import tpu as pltpu
from jax.experimental.pallas import tpu_sc as plsc

N = 10000
NPAD = 10240
NW = 32
CK = 128
H = 64
RPT_DEG = NPAD // 16
RPT_AGG = (NPAD // 2) // 16


def _mesh():
    return plsc.VectorSubcoreMesh(core_axis_name="c", subcore_axis_name="s")


def _deg_body(dsts_hbm, const_hbm, out_hbm, idx_dst, ones_v, acc, sem):
    c = lax.axis_index("c")
    s = lax.axis_index("s")
    wid = s * 2 + c
    nchunks = dsts_hbm.shape[1]
    pltpu.sync_copy(const_hbm.at[pl.ds(0, CK)], ones_v)
    pltpu.sync_copy(dsts_hbm.at[wid], idx_dst)
    pltpu.sync_copy(
        const_hbm.at[pl.ds(CK, RPT_DEG)],
        acc.at[pl.ds(s * RPT_DEG, RPT_DEG)],
    )
    plsc.subcore_barrier()

    def step(j, _):
        pltpu.async_copy(ones_v, acc.at[idx_dst.at[j]], sem, add=True)
        return 0

    lax.fori_loop(0, nchunks, step, 0)

    def drain(j, _):
        pltpu.make_async_copy(ones_v, acc.at[idx_dst.at[j]], sem).wait()
        return 0

    lax.fori_loop(0, nchunks, drain, 0)
    plsc.subcore_barrier()
    pltpu.sync_copy(
        acc.at[pl.ds(s * RPT_DEG, RPT_DEG)],
        out_hbm.at[c].at[pl.ds(s * RPT_DEG, RPT_DEG)],
    )


def _sc_deg(dsts, const16):
    kern = pl.kernel(
        _deg_body,
        out_type=jax.ShapeDtypeStruct((2, NPAD, 16), jnp.float32),
        mesh=_mesh(),
        scratch_types=[
            pltpu.VMEM(dsts.shape[1:], jnp.int32),
            pltpu.VMEM((CK, 16), jnp.float32),
            pltpu.VMEM_SHARED((NPAD, 16), jnp.float32),
            pltpu.SemaphoreType.DMA,
        ],
    )
    return kern(dsts, const16)


NBUF = 8
LOOK = 4


def _agg_body(tbl_hbm, gsrcs_hbm, sdsts_hbm, zeros_hbm, out_hbm,
              idx_src, idx_dst, rows, acc, *sems):
    c = lax.axis_index("c")
    s = lax.axis_index("s")
    wid = s * 2 + c
    nchunks = gsrcs_hbm.shape[1]
    gsems, ssems = sems[:NBUF], sems[NBUF:]
    pltpu.sync_copy(gsrcs_hbm.at[wid], idx_src)
    pltpu.sync_copy(sdsts_hbm.at[wid], idx_dst)
    pltpu.sync_copy(zeros_hbm, acc.at[pl.ds(s * RPT_DEG, RPT_DEG)])
    plsc.subcore_barrier()

    def gather(j, b):
        pltpu.async_copy(tbl_hbm.at[idx_src.at[j]], rows.at[b], gsems[b])

    def wait_gather(j, b):
        pltpu.make_async_copy(tbl_hbm.at[idx_src.at[j]], rows.at[b],
                              gsems[b]).wait()

    def scatter(j, b):
        pltpu.async_copy(rows.at[b], acc.at[idx_dst.at[j]], ssems[b],
                         add=True)

    def wait_scatter(j, b):
        pltpu.make_async_copy(rows.at[b], acc.at[idx_dst.at[j]],
                              ssems[b]).wait()

    for b in range(LOOK):
        gather(b, b)

    def outer(i, _):
        j0 = NBUF * i
        for b in range(NBUF):
            j = j0 + b
            wait_gather(j, b)
            scatter(j, b)
            bn = (b + LOOK) % NBUF
            @pl.when(j + LOOK < nchunks)
            def _(j=j, bn=bn):
                @pl.when(j - LOOK >= 0)
                def _():
                    wait_scatter(j - LOOK, bn)
                gather(j + LOOK, bn)
        return 0

    lax.fori_loop(0, nchunks // NBUF, outer, 0)
    for k in range(NBUF):
        j = nchunks - NBUF + k
        wait_scatter(j, j % NBUF)
    plsc.subcore_barrier()
    pltpu.sync_copy(
        acc.at[pl.ds(s * RPT_DEG, RPT_DEG)],
        out_hbm.at[c].at[pl.ds(s * RPT_DEG, RPT_DEG)],
    )


def _sc_agg(tbl, gsrcs, sdsts, zeros):
    kern = pl.kernel(
        _agg_body,
        out_type=jax.ShapeDtypeStruct((2, NPAD, H), jnp.float32),
        mesh=_mesh(),
        scratch_types=[
            pltpu.VMEM(gsrcs.shape[1:], jnp.int32),
            pltpu.VMEM(sdsts.shape[1:], jnp.int32),
            pltpu.VMEM((NBUF, CK, H), jnp.float32),
            pltpu.VMEM_SHARED((NPAD, H), jnp.float32),
        ] + [pltpu.SemaphoreType.DMA] * (2 * NBUF),
        compiler_params=pltpu.CompilerParams(use_tc_tiling_on_sc=False),
    )
    return kern(tbl, gsrcs, sdsts, zeros)


def _decode_body(zi_ref, zj_ref, out_ref):
    logits = jax.lax.dot_general(
        zi_ref[...], zj_ref[...],
        (((1,), (1,)), ((), ())),
        preferred_element_type=jnp.float32,
    )
    out_ref[...] = 0.5 * (1.0 + jnp.tanh(logits * 0.5))


@functools.partial(jax.jit, static_argnames=("bm",))
def _decode(z, bm=256):
    n, cdim = z.shape
    grid = (pl.cdiv(n, bm),)
    return pl.pallas_call(
        _decode_body,
        grid=grid,
        in_specs=[
            pl.BlockSpec((bm, cdim), lambda i: (i, 0)),
            pl.BlockSpec((n, cdim), lambda i: (0, 0)),
        ],
        out_specs=pl.BlockSpec((bm, n), lambda i: (i, 0)),
        out_shape=jax.ShapeDtypeStruct((n, n), jnp.float32),
    )(z, z)


def _prep_idx(idx, pad_base, n_pad_zone_rows=32):
    e = idx.shape[0]
    epw = -(-e // NW)
    nchunks = -(-epw // CK)
    nchunks = -(-nchunks // NBUF) * NBUF
    total = NW * nchunks * CK
    pad = jnp.asarray(
        pad_base + (np.arange(total - e) % n_pad_zone_rows), jnp.int32)
    return jnp.concatenate([idx.astype(jnp.int32), pad]).reshape(
        NW, nchunks, CK)


def kernel(x, edge_index, eps, W1, b1, Wmu, bmu, Wls, bls):
    src, dst = edge_index[0], edge_index[1]
    src = src.astype(jnp.int32)
    dst = dst.astype(jnp.int32)
    dsts = _prep_idx(dst, N)
    gsrcs = _prep_idx(src, 0)
    sdsts = dsts
    zeros = jnp.zeros((RPT_DEG, H), jnp.float32)
    const16 = jnp.concatenate(
        [jnp.ones((CK, 16), jnp.float32),
         jnp.zeros((RPT_DEG, 16), jnp.float32)])

    degp = _sc_deg(dsts, const16)
    deg = degp[0, :N, 0] + degp[1, :N, 0] + 1.0
    dinv = jax.lax.rsqrt(deg)

    def prop(g):
        parts = _sc_agg(g, gsrcs, sdsts, zeros)
        return parts[0, :N] + parts[1, :N] + g

    g1 = (x @ W1) * dinv[:, None]
    h = jax.nn.relu(prop(g1) * dinv[:, None] + b1)

    g2 = h * dinv[:, None]
    q = prop(g2) * dinv[:, None]

    mu = q @ Wmu + bmu
    logstd = q @ Wls + bls
    z = mu + eps * jnp.exp(logstd)
    adj = _decode(z)
    return adj, mu, logstd
